# Initial kernel scaffold; baseline (speedup 1.0000x reference)
import jax
import jax.numpy as jnp
from jax.experimental import pallas as pl


def kernel(x, v, tables, sw1, sb1, sw2, sb2, rw1, rb1, rw2, rb2, rw3, rb3):
    raise NotImplementedError("write your pallas kernel here")


# speedup vs baseline: 10.0156x; 10.0156x over previous
import functools

import numpy as np
import jax
import jax.numpy as jnp
from jax import lax
from jax.experimental import pallas as pl
from jax.experimental.pallas import tpu as pltpu
from jax.experimental.pallas import tpu_sc as plsc

L = 16
F = 2
T = 1 << 19
N_MIN = 16
N_MAX = 2048
NPTS = 131072
_BG = float(np.exp((np.log(N_MAX) - np.log(N_MIN)) / (L - 1)))
RES = [int(np.floor(N_MIN * (_BG ** l))) for l in range(L)]
P2 = np.int32(2654435761 - (1 << 32))
P3 = np.int32(805459861)
MASK = np.int32(T - 1)

_info = plsc.get_sparse_core_info()
NC = _info.num_cores
NS = _info.num_subcores
NW = NC * NS
PPW = NPTS // NW
BLK = 128
NBLK = PPW // BLK
GRP = BLK // 16
NIDX = BLK * 8
STR = 128
NSTR = (NIDX * F) // STR


def _frac_parts(xyz, off, res):
    xv = xyz[0, pl.ds(off, 16)]
    yv = xyz[1, pl.ds(off, 16)]
    zv = xyz[2, pl.ds(off, 16)]
    xs = xv * res
    ys = yv * res
    zs = zv * res
    xi = xs.astype(jnp.int32)
    yi = ys.astype(jnp.int32)
    zi = zs.astype(jnp.int32)
    fx = xs - xi.astype(jnp.float32)
    fy = ys - yi.astype(jnp.float32)
    fz = zs - zi.astype(jnp.float32)
    return xi, yi, zi, fx, fy, fz


def _make_encoder():
    mesh = plsc.VectorSubcoreMesh(core_axis_name="c", subcore_axis_name="s")

    @functools.partial(
        pl.kernel,
        mesh=mesh,
        out_type=jax.ShapeDtypeStruct((2 * L, NPTS), jnp.float32),
        scratch_types=[
            pltpu.VMEM((3, BLK), jnp.float32),
            pltpu.VMEM((NIDX * F,), jnp.int32),
            pltpu.VMEM((NIDX * F,), jnp.int32),
            pltpu.VMEM((NIDX * F,), jnp.float32),
            pltpu.VMEM((NIDX * F,), jnp.float32),
            pltpu.VMEM((2 * L, BLK), jnp.float32),
            pltpu.SemaphoreType.DMA,
            pltpu.SemaphoreType.DMA,
        ],
    )
    def enc(xt, tabs, out, xyz, ib0, ib1, fb0, fb1, ob, sema, semb):
        wid = lax.axis_index("s") * NC + lax.axis_index("c")
        ibufs = [ib0, ib1]
        fbufs = [fb0, fb1]
        sems = [sema, semb]

        def hash_level(l, ib):
            res = float(RES[l])
            base_l = np.int32(l * T)

            def g_body(g, carry):
                xi, yi, zi, fx, fy, fz = _frac_parts(xyz, g * 16, res)
                hy0 = yi * P2
                hy1 = hy0 + P2
                hz0 = zi * P3
                hz1 = hz0 + P3
                e00 = hy0 ^ hz0
                e01 = hy0 ^ hz1
                e10 = hy1 ^ hz0
                e11 = hy1 ^ hz1
                x1 = xi + 1
                hs = (xi ^ e00, xi ^ e01, xi ^ e10, xi ^ e11,
                      x1 ^ e00, x1 ^ e01, x1 ^ e10, x1 ^ e11)
                base = g * 128
                for c in range(8):
                    d = (((hs[c] & MASK) + base_l) << 1)
                    ib[pl.ds(base + c * 16, 16)] = d
                    ib[pl.ds(NIDX + base + c * 16, 16)] = d + 1
                return carry

            lax.fori_loop(0, GRP, g_body, None)

        def fire(ib, fb, sem):
            return [pltpu.async_copy(
                tabs.at[ib.at[pl.ds(j * STR, STR)]],
                fb.at[pl.ds(j * STR, STR)], sem) for j in range(NSTR)]

        def interp(l, fb):
            res = float(RES[l])

            def g_body(g, carry):
                _, _, _, fx, fy, fz = _frac_parts(xyz, g * 16, res)
                wx0 = 1.0 - fx
                wy0 = 1.0 - fy
                wz0 = 1.0 - fz
                wxy = (wx0 * wy0, wx0 * fy, fx * wy0, fx * fy)
                acc0 = jnp.zeros((16,), jnp.float32)
                acc1 = jnp.zeros((16,), jnp.float32)
                base = g * 128
                for c in range(8):
                    w = wxy[c >> 1] * (fz if (c & 1) else wz0)
                    f0 = fb[pl.ds(base + c * 16, 16)]
                    f1 = fb[pl.ds(NIDX + base + c * 16, 16)]
                    acc0 = acc0 + w * f0
                    acc1 = acc1 + w * f1
                ob[2 * l, pl.ds(g * 16, 16)] = acc0
                ob[2 * l + 1, pl.ds(g * 16, 16)] = acc1
                return carry

            lax.fori_loop(0, GRP, g_body, None)

        def block(b, carry):
            col = wid * PPW + b * BLK
            pltpu.sync_copy(xt.at[:, pl.ds(col, BLK)], xyz)
            hash_level(0, ibufs[0])
            prev = fire(ibufs[0], fbufs[0], sems[0])
            for l in range(1, L):
                cb = l % 2
                pb = (l - 1) % 2
                hash_level(l, ibufs[cb])
                cur = fire(ibufs[cb], fbufs[cb], sems[cb])
                for cp in prev:
                    cp.wait()
                interp(l - 1, fbufs[pb])
                prev = cur
            for cp in prev:
                cp.wait()
            interp(L - 1, fbufs[(L - 1) % 2])
            pltpu.sync_copy(ob, out.at[:, pl.ds(col, BLK)])
            return carry

        lax.fori_loop(0, NBLK, block, None)

    return enc


_encode = _make_encoder()

TB = 1024
_HI = lax.Precision.HIGHEST


def _dot(a, b):
    return lax.dot_general(a, b, (((1,), (0,)), ((), ())),
                           precision=_HI, preferred_element_type=jnp.float32)


def _mlp_body(f_ref, v_ref, sw1_ref, sb1_ref, sw2_ref, sb2_ref, rw1_ref,
              rw1x_ref, rb1_ref, rw2_ref, rb2_ref, rw3_ref, rb3_ref,
              sig_ref, rgb_ref):
    f = f_ref[...]
    h1 = lax.dot_general(f, sw1_ref[...], (((0,), (0,)), ((), ())),
                         precision=_HI, preferred_element_type=jnp.float32)
    h1 = jnp.maximum(h1 + sb1_ref[...], 0.0)
    out = _dot(h1, sw2_ref[...]) + sb2_ref[...]
    sig_ref[...] = out[:, 0:1]

    vv = v_ref[...]
    rw1 = rw1_ref[...]
    r = _dot(out, rw1x_ref[...]) + rb1_ref[...]
    r = r + _dot(vv, rw1[0:3, :])
    for k in range(4):
        s = float(2.0 ** k)
        r = r + _dot(jnp.sin(s * vv), rw1[3 + 6 * k:6 + 6 * k, :])
        r = r + _dot(jnp.cos(s * vv), rw1[6 + 6 * k:9 + 6 * k, :])
    h = jnp.maximum(r, 0.0)
    h = jnp.maximum(_dot(h, rw2_ref[...]) + rb2_ref[...], 0.0)
    o = _dot(h, rw3_ref[...]) + rb3_ref[...]
    rgb_ref[...] = 1.0 / (1.0 + jnp.exp(-o))


def _full(shape):
    return pl.BlockSpec(shape, lambda i: (0, 0))


_mlp = pl.pallas_call(
    _mlp_body,
    grid=(NPTS // TB,),
    in_specs=[
        pl.BlockSpec((2 * L, TB), lambda i: (0, i)),
        pl.BlockSpec((TB, 3), lambda i: (i, 0)),
        _full((2 * L, 64)),
        _full((1, 64)),
        _full((64, 16)),
        _full((1, 16)),
        _full((27, 64)),
        _full((16, 64)),
        _full((1, 64)),
        _full((64, 64)),
        _full((1, 64)),
        _full((64, 3)),
        _full((1, 3)),
    ],
    out_specs=[
        pl.BlockSpec((TB, 1), lambda i: (i, 0)),
        pl.BlockSpec((TB, 3), lambda i: (i, 0)),
    ],
    out_shape=[
        jax.ShapeDtypeStruct((NPTS, 1), jnp.float32),
        jax.ShapeDtypeStruct((NPTS, 3), jnp.float32),
    ],
)


def kernel(x, v, tables, sw1, sb1, sw2, sb2, rw1, rb1, rw2, rb2, rw3, rb3):
    xt = x.T
    tabs = tables.reshape(-1)
    feats = _encode(xt, tabs)
    rw1x = jnp.concatenate(
        [jnp.zeros((1, 64), jnp.float32), rw1[27:42, :]], axis=0)
    sig, rgb = _mlp(feats, v, sw1, sb1.reshape(1, -1), sw2,
                    sb2.reshape(1, -1), rw1[0:27, :], rw1x,
                    rb1.reshape(1, -1), rw2, rb2.reshape(1, -1), rw3,
                    rb3.reshape(1, -1))
    return sig.reshape(-1), rgb


# speedup vs baseline: 43.9105x; 4.3842x over previous
import functools

import numpy as np
import jax
import jax.numpy as jnp
from jax import lax
from jax.experimental import pallas as pl
from jax.experimental.pallas import tpu as pltpu
from jax.experimental.pallas import tpu_sc as plsc

L = 16
F = 2
T = 1 << 19
N_MIN = 16
N_MAX = 2048
NPTS = 131072
_BG = float(np.exp((np.log(N_MAX) - np.log(N_MIN)) / (L - 1)))
RES = [int(np.floor(N_MIN * (_BG ** l))) for l in range(L)]
P2 = np.int32(2654435761 - (1 << 32))
P3 = np.int32(805459861)
MASK = np.int32(T - 1)

_info = plsc.get_sparse_core_info()
NC = _info.num_cores
NS = _info.num_subcores
NW = NC * NS
PPW = NPTS // NW
BLK = 128
NBLK = PPW // BLK
GRP = BLK // 16
NIDX = BLK * 8
STR = 128
NSTR = (NIDX * F) // STR


def _frac_parts(xyz, off, res):
    xv = xyz[0, pl.ds(off, 16)]
    yv = xyz[1, pl.ds(off, 16)]
    zv = xyz[2, pl.ds(off, 16)]
    xs = xv * res
    ys = yv * res
    zs = zv * res
    xi = xs.astype(jnp.int32)
    yi = ys.astype(jnp.int32)
    zi = zs.astype(jnp.int32)
    fx = xs - xi.astype(jnp.float32)
    fy = ys - yi.astype(jnp.float32)
    fz = zs - zi.astype(jnp.float32)
    return xi, yi, zi, fx, fy, fz


def _make_encoder():
    mesh = plsc.VectorSubcoreMesh(core_axis_name="c", subcore_axis_name="s")

    @functools.partial(
        pl.kernel,
        mesh=mesh,
        out_type=jax.ShapeDtypeStruct((2 * L, NPTS), jnp.float32),
        scratch_types=[
            pltpu.VMEM((3, BLK), jnp.float32),
            pltpu.VMEM((NIDX * F,), jnp.int32),
            pltpu.VMEM((NIDX * F,), jnp.int32),
            pltpu.VMEM((NIDX * F,), jnp.float32),
            pltpu.VMEM((NIDX * F,), jnp.float32),
            pltpu.VMEM((2 * L, BLK), jnp.float32),
            pltpu.SemaphoreType.DMA,
            pltpu.SemaphoreType.DMA,
        ],
    )
    def enc(xt, tabs, out, xyz, ib0, ib1, fb0, fb1, ob, sema, semb):
        wid = lax.axis_index("s") * NC + lax.axis_index("c")
        ibufs = [ib0, ib1]
        fbufs = [fb0, fb1]
        sems = [sema, semb]

        def hash_level(l, ib):
            res = float(RES[l])
            base_l = np.int32(l * T * F)

            def g_body(g, carry):
                xi, yi, zi, fx, fy, fz = _frac_parts(xyz, g * 16, res)
                hy0 = yi * P2
                hy1 = hy0 + P2
                hz0 = zi * P3
                hz1 = hz0 + P3
                e00 = hy0 ^ hz0
                e01 = hy0 ^ hz1
                e10 = hy1 ^ hz0
                e11 = hy1 ^ hz1
                x1 = xi + 1
                hs = (xi ^ e00, xi ^ e01, xi ^ e10, xi ^ e11,
                      x1 ^ e00, x1 ^ e01, x1 ^ e10, x1 ^ e11)
                base = g * 128
                for c in range(8):
                    t = hs[c] & MASK
                    d = base_l + (((t >> 7) << 8) | (t & 127))
                    ib[pl.ds(base + c * 16, 16)] = d
                    ib[pl.ds(NIDX + base + c * 16, 16)] = d + 128
                return carry

            lax.fori_loop(0, GRP, g_body, None)

        def fire(ib, fb, sem):
            return [pltpu.async_copy(
                tabs.at[ib.at[pl.ds(j * STR, STR)]],
                fb.at[pl.ds(j * STR, STR)], sem) for j in range(NSTR)]

        def interp(l, fb):
            res = float(RES[l])

            def g_body(g, carry):
                _, _, _, fx, fy, fz = _frac_parts(xyz, g * 16, res)
                wx0 = 1.0 - fx
                wy0 = 1.0 - fy
                wz0 = 1.0 - fz
                wxy = (wx0 * wy0, wx0 * fy, fx * wy0, fx * fy)
                acc0 = jnp.zeros((16,), jnp.float32)
                acc1 = jnp.zeros((16,), jnp.float32)
                base = g * 128
                for c in range(8):
                    w = wxy[c >> 1] * (fz if (c & 1) else wz0)
                    f0 = fb[pl.ds(base + c * 16, 16)]
                    f1 = fb[pl.ds(NIDX + base + c * 16, 16)]
                    acc0 = acc0 + w * f0
                    acc1 = acc1 + w * f1
                ob[2 * l, pl.ds(g * 16, 16)] = acc0
                ob[2 * l + 1, pl.ds(g * 16, 16)] = acc1
                return carry

            lax.fori_loop(0, GRP, g_body, None)

        def block(b, carry):
            col = wid * PPW + b * BLK
            pltpu.sync_copy(xt.at[:, pl.ds(col, BLK)], xyz)
            hash_level(0, ibufs[0])
            prev = fire(ibufs[0], fbufs[0], sems[0])
            for l in range(1, L):
                cb = l % 2
                pb = (l - 1) % 2
                hash_level(l, ibufs[cb])
                cur = fire(ibufs[cb], fbufs[cb], sems[cb])
                for cp in prev:
                    cp.wait()
                interp(l - 1, fbufs[pb])
                prev = cur
            for cp in prev:
                cp.wait()
            interp(L - 1, fbufs[(L - 1) % 2])
            pltpu.sync_copy(ob, out.at[:, pl.ds(col, BLK)])
            return carry

        lax.fori_loop(0, NBLK, block, None)

    return enc


_encode = _make_encoder()

TB = 1024
_HI = lax.Precision.HIGHEST


def _dot(a, b):
    return lax.dot_general(a, b, (((1,), (0,)), ((), ())),
                           precision=_HI, preferred_element_type=jnp.float32)


def _mlp_body(f_ref, v_ref, sw1_ref, sb1_ref, sw2_ref, sb2_ref, rw1_ref,
              rw1x_ref, rb1_ref, rw2_ref, rb2_ref, rw3_ref, rb3_ref,
              sig_ref, rgb_ref):
    f = f_ref[...]
    h1 = lax.dot_general(f, sw1_ref[...], (((0,), (0,)), ((), ())),
                         precision=_HI, preferred_element_type=jnp.float32)
    h1 = jnp.maximum(h1 + sb1_ref[...], 0.0)
    out = _dot(h1, sw2_ref[...]) + sb2_ref[...]
    sig_ref[...] = out[:, 0:1]

    vv = v_ref[...]
    rw1 = rw1_ref[...]
    r = _dot(out, rw1x_ref[...]) + rb1_ref[...]
    r = r + _dot(vv, rw1[0:3, :])
    for k in range(4):
        s = float(2.0 ** k)
        r = r + _dot(jnp.sin(s * vv), rw1[3 + 6 * k:6 + 6 * k, :])
        r = r + _dot(jnp.cos(s * vv), rw1[6 + 6 * k:9 + 6 * k, :])
    h = jnp.maximum(r, 0.0)
    h = jnp.maximum(_dot(h, rw2_ref[...]) + rb2_ref[...], 0.0)
    o = _dot(h, rw3_ref[...]) + rb3_ref[...]
    rgb_ref[...] = 1.0 / (1.0 + jnp.exp(-o))


def _full(shape):
    return pl.BlockSpec(shape, lambda i: (0, 0))


_mlp = pl.pallas_call(
    _mlp_body,
    grid=(NPTS // TB,),
    in_specs=[
        pl.BlockSpec((2 * L, TB), lambda i: (0, i)),
        pl.BlockSpec((TB, 3), lambda i: (i, 0)),
        _full((2 * L, 64)),
        _full((1, 64)),
        _full((64, 16)),
        _full((1, 16)),
        _full((27, 64)),
        _full((16, 64)),
        _full((1, 64)),
        _full((64, 64)),
        _full((1, 64)),
        _full((64, 3)),
        _full((1, 3)),
    ],
    out_specs=[
        pl.BlockSpec((TB, 1), lambda i: (i, 0)),
        pl.BlockSpec((TB, 3), lambda i: (i, 0)),
    ],
    out_shape=[
        jax.ShapeDtypeStruct((NPTS, 1), jnp.float32),
        jax.ShapeDtypeStruct((NPTS, 3), jnp.float32),
    ],
)


def kernel(x, v, tables, sw1, sb1, sw2, sb2, rw1, rb1, rw2, rb2, rw3, rb3):
    xt = x.T
    tabs = tables.reshape(L, T // 128, 128, F).transpose(0, 1, 3, 2).reshape(-1)
    feats = _encode(xt, tabs)
    rw1x = jnp.concatenate(
        [jnp.zeros((1, 64), jnp.float32), rw1[27:42, :]], axis=0)
    sig, rgb = _mlp(feats, v, sw1, sb1.reshape(1, -1), sw2,
                    sb2.reshape(1, -1), rw1[0:27, :], rw1x,
                    rb1.reshape(1, -1), rw2, rb2.reshape(1, -1), rw3,
                    rb3.reshape(1, -1))
    return sig.reshape(-1), rgb


# speedup vs baseline: 102.0122x; 2.3232x over previous
import functools

import numpy as np
import jax
import jax.numpy as jnp
from jax import lax
from jax.experimental import pallas as pl
from jax.experimental.pallas import tpu as pltpu
from jax.experimental.pallas import tpu_sc as plsc

L = 16
F = 2
T = 1 << 19
N_MIN = 16
N_MAX = 2048
NPTS = 131072
_BG = float(np.exp((np.log(N_MAX) - np.log(N_MIN)) / (L - 1)))
RES = [int(np.floor(N_MIN * (_BG ** l))) for l in range(L)]
P2 = np.int32(2654435761 - (1 << 32))
P3 = np.int32(805459861)
MASK = np.int32(T - 1)

_info = plsc.get_sparse_core_info()
NC = _info.num_cores
NS = _info.num_subcores
NW = NC * NS
PPW = NPTS // NW
BLK = 128
NBLK = PPW // BLK
GRP = BLK // 16
NIDX = BLK * 8
STR = 128
NSTR = (NIDX * F) // STR


def _frac_parts(xyz, off, res):
    xv = xyz[0, pl.ds(off, 16)]
    yv = xyz[1, pl.ds(off, 16)]
    zv = xyz[2, pl.ds(off, 16)]
    xs = xv * res
    ys = yv * res
    zs = zv * res
    xi = xs.astype(jnp.int32)
    yi = ys.astype(jnp.int32)
    zi = zs.astype(jnp.int32)
    fx = xs - xi.astype(jnp.float32)
    fy = ys - yi.astype(jnp.float32)
    fz = zs - zi.astype(jnp.float32)
    return xi, yi, zi, fx, fy, fz


def _make_encoder():
    mesh = plsc.VectorSubcoreMesh(core_axis_name="c", subcore_axis_name="s")

    @functools.partial(
        pl.kernel,
        mesh=mesh,
        out_type=jax.ShapeDtypeStruct((2 * L, NPTS), jnp.float32),
        scratch_types=[
            pltpu.VMEM((3, BLK), jnp.float32),
            pltpu.VMEM((NIDX * F,), jnp.int32),
            pltpu.VMEM((NIDX * F,), jnp.int32),
            pltpu.VMEM((NIDX * F,), jnp.float32),
            pltpu.VMEM((NIDX * F,), jnp.float32),
            pltpu.VMEM((2 * L, BLK), jnp.float32),
            pltpu.SemaphoreType.DMA,
            pltpu.SemaphoreType.DMA,
        ],
    )
    def enc(xt, tabs, out, xyz, ib0, ib1, fb0, fb1, ob, sema, semb):
        wid = lax.axis_index("s") * NC + lax.axis_index("c")
        ibufs = [ib0, ib1]
        fbufs = [fb0, fb1]
        sems = [sema, semb]

        def hash_level(l, ib):
            res = float(RES[l])
            base_l = np.int32(l * T * F)

            def g_body(g, carry):
                xi, yi, zi, fx, fy, fz = _frac_parts(xyz, g * 16, res)
                hy0 = yi * P2
                hy1 = hy0 + P2
                hz0 = zi * P3
                hz1 = hz0 + P3
                e00 = hy0 ^ hz0
                e01 = hy0 ^ hz1
                e10 = hy1 ^ hz0
                e11 = hy1 ^ hz1
                x1 = xi + 1
                hs = (xi ^ e00, xi ^ e01, xi ^ e10, xi ^ e11,
                      x1 ^ e00, x1 ^ e01, x1 ^ e10, x1 ^ e11)
                base = g * 128
                for c in range(8):
                    t = hs[c] & MASK
                    d = base_l + (((t >> 7) << 8) | (t & 127))
                    ib[pl.ds(base + c * 16, 16)] = d
                    ib[pl.ds(NIDX + base + c * 16, 16)] = d + 128
                return carry

            lax.fori_loop(0, GRP, g_body, None)

        def fire(ib, fb, sem):
            return [pltpu.async_copy(
                tabs.at[ib.at[pl.ds(j * STR, STR)]],
                fb.at[pl.ds(j * STR, STR)], sem) for j in range(NSTR)]

        def interp(l, fb):
            res = float(RES[l])

            def g_body(g, carry):
                _, _, _, fx, fy, fz = _frac_parts(xyz, g * 16, res)
                wx0 = 1.0 - fx
                wy0 = 1.0 - fy
                wz0 = 1.0 - fz
                wxy = (wx0 * wy0, wx0 * fy, fx * wy0, fx * fy)
                acc0 = jnp.zeros((16,), jnp.float32)
                acc1 = jnp.zeros((16,), jnp.float32)
                base = g * 128
                for c in range(8):
                    w = wxy[c >> 1] * (fz if (c & 1) else wz0)
                    f0 = fb[pl.ds(base + c * 16, 16)]
                    f1 = fb[pl.ds(NIDX + base + c * 16, 16)]
                    acc0 = acc0 + w * f0
                    acc1 = acc1 + w * f1
                ob[2 * l, pl.ds(g * 16, 16)] = acc0
                ob[2 * l + 1, pl.ds(g * 16, 16)] = acc1
                return carry

            lax.fori_loop(0, GRP, g_body, None)

        def block(b, carry):
            col = wid * PPW + b * BLK
            pltpu.sync_copy(xt.at[:, pl.ds(col, BLK)], xyz)
            hash_level(0, ibufs[0])
            prev = fire(ibufs[0], fbufs[0], sems[0])
            for l in range(1, L):
                cb = l % 2
                pb = (l - 1) % 2
                hash_level(l, ibufs[cb])
                cur = fire(ibufs[cb], fbufs[cb], sems[cb])
                for cp in prev:
                    cp.wait()
                interp(l - 1, fbufs[pb])
                prev = cur
            for cp in prev:
                cp.wait()
            interp(L - 1, fbufs[(L - 1) % 2])
            pltpu.sync_copy(ob, out.at[:, pl.ds(col, BLK)])
            return carry

        lax.fori_loop(0, NBLK, block, None)

    return enc


_encode = _make_encoder()

TB = 2048


def _dot(a, b):
    return lax.dot_general(a, b, (((1,), (0,)), ((), ())),
                           preferred_element_type=jnp.float32)


def _mlp_body(f_ref, vt_ref, w1t_ref, b1_ref, w2t_ref, b2_ref, rwxt_ref,
              emat_ref, rwvt_ref, rwst_ref, rwct_ref, rb1_ref, rw2t_ref,
              rb2_ref, rw3t_ref, rb3_ref, sig_ref, rgb_ref):
    f = f_ref[...]
    h1 = jnp.maximum(_dot(w1t_ref[...], f) + b1_ref[...], 0.0)
    out = _dot(w2t_ref[...], h1) + b2_ref[...]
    sig_ref[...] = out[0:1, :]

    vt = vt_ref[...]
    v12 = _dot(emat_ref[...], vt)
    r = _dot(rwxt_ref[...], out) + rb1_ref[...]
    r = r + _dot(rwvt_ref[...], vt)
    r = r + _dot(rwst_ref[...], jnp.sin(v12))
    r = r + _dot(rwct_ref[...], jnp.cos(v12))
    h = jnp.maximum(r, 0.0)
    h = jnp.maximum(_dot(rw2t_ref[...], h) + rb2_ref[...], 0.0)
    o = _dot(rw3t_ref[...], h) + rb3_ref[...]
    rgb_ref[...] = 1.0 / (1.0 + jnp.exp(-o))


def _full(shape):
    return pl.BlockSpec(shape, lambda i: (0, 0))


_mlp = pl.pallas_call(
    _mlp_body,
    grid=(NPTS // TB,),
    in_specs=[
        pl.BlockSpec((2 * L, TB), lambda i: (0, i)),
        pl.BlockSpec((3, TB), lambda i: (0, i)),
        _full((64, 2 * L)),
        _full((64, 1)),
        _full((16, 64)),
        _full((16, 1)),
        _full((64, 16)),
        _full((12, 3)),
        _full((64, 3)),
        _full((64, 12)),
        _full((64, 12)),
        _full((64, 1)),
        _full((64, 64)),
        _full((64, 1)),
        _full((3, 64)),
        _full((3, 1)),
    ],
    out_specs=[
        pl.BlockSpec((1, TB), lambda i: (0, i)),
        pl.BlockSpec((3, TB), lambda i: (0, i)),
    ],
    out_shape=[
        jax.ShapeDtypeStruct((1, NPTS), jnp.float32),
        jax.ShapeDtypeStruct((3, NPTS), jnp.float32),
    ],
)

_SIN_ROWS = np.array([3 + 6 * k + d for k in range(4) for d in range(3)])
_COS_ROWS = _SIN_ROWS + 3
_EMAT = np.zeros((12, 3), np.float32)
for _k in range(4):
    for _d in range(3):
        _EMAT[3 * _k + _d, _d] = float(2.0 ** _k)


def kernel(x, v, tables, sw1, sb1, sw2, sb2, rw1, rb1, rw2, rb2, rw3, rb3):
    xt = x.T
    tabs = tables.reshape(L, T // 128, 128, F).transpose(0, 1, 3, 2).reshape(-1)
    feats = _encode(xt, tabs)
    rwx = jnp.concatenate(
        [jnp.zeros((1, 64), jnp.float32), rw1[27:42, :]], axis=0)
    sig, rgbt = _mlp(
        feats, v.T, sw1.T, sb1.reshape(-1, 1), sw2.T, sb2.reshape(-1, 1),
        rwx.T, jnp.asarray(_EMAT), rw1[0:3, :].T,
        jnp.take(rw1, _SIN_ROWS, axis=0).T,
        jnp.take(rw1, _COS_ROWS, axis=0).T,
        rb1.reshape(-1, 1), rw2.T, rb2.reshape(-1, 1), rw3.T,
        rb3.reshape(-1, 1))
    return sig.reshape(-1), rgbt.T


# speedup vs baseline: 102.3102x; 1.0029x over previous
import functools

import numpy as np
import jax
import jax.numpy as jnp
from jax import lax
from jax.experimental import pallas as pl
from jax.experimental.pallas import tpu as pltpu
from jax.experimental.pallas import tpu_sc as plsc

L = 16
F = 2
T = 1 << 19
N_MIN = 16
N_MAX = 2048
NPTS = 131072
_BG = float(np.exp((np.log(N_MAX) - np.log(N_MIN)) / (L - 1)))
RES = [int(np.floor(N_MIN * (_BG ** l))) for l in range(L)]
P2 = np.int32(2654435761 - (1 << 32))
P3 = np.int32(805459861)
MASK = np.int32(T - 1)

_info = plsc.get_sparse_core_info()
NC = _info.num_cores
NS = _info.num_subcores
NW = NC * NS
PPW = NPTS // NW
BLK = 128
NBLK = PPW // BLK
GRP = BLK // 16
NIDX = BLK * 8
STR = 512
NSTR = (NIDX * F) // STR


def _frac_parts(xyz, off, res):
    xv = xyz[0, pl.ds(off, 16)]
    yv = xyz[1, pl.ds(off, 16)]
    zv = xyz[2, pl.ds(off, 16)]
    xs = xv * res
    ys = yv * res
    zs = zv * res
    xi = xs.astype(jnp.int32)
    yi = ys.astype(jnp.int32)
    zi = zs.astype(jnp.int32)
    fx = xs - xi.astype(jnp.float32)
    fy = ys - yi.astype(jnp.float32)
    fz = zs - zi.astype(jnp.float32)
    return xi, yi, zi, fx, fy, fz


def _make_encoder():
    mesh = plsc.VectorSubcoreMesh(core_axis_name="c", subcore_axis_name="s")

    @functools.partial(
        pl.kernel,
        mesh=mesh,
        out_type=jax.ShapeDtypeStruct((2 * L, NPTS), jnp.float32),
        scratch_types=[
            pltpu.VMEM((3, BLK), jnp.float32),
            pltpu.VMEM((NIDX * F,), jnp.int32),
            pltpu.VMEM((NIDX * F,), jnp.int32),
            pltpu.VMEM((NIDX * F,), jnp.float32),
            pltpu.VMEM((NIDX * F,), jnp.float32),
            pltpu.VMEM((2 * L, BLK), jnp.float32),
            pltpu.SemaphoreType.DMA,
            pltpu.SemaphoreType.DMA,
        ],
    )
    def enc(xt, tabs, out, xyz, ib0, ib1, fb0, fb1, ob, sema, semb):
        wid = lax.axis_index("s") * NC + lax.axis_index("c")
        ibufs = [ib0, ib1]
        fbufs = [fb0, fb1]
        sems = [sema, semb]

        def hash_level(l, ib):
            res = float(RES[l])
            base_l = np.int32(l * T * F)

            def g_body(g, carry):
                xi, yi, zi, fx, fy, fz = _frac_parts(xyz, g * 16, res)
                hy0 = yi * P2
                hy1 = hy0 + P2
                hz0 = zi * P3
                hz1 = hz0 + P3
                e00 = hy0 ^ hz0
                e01 = hy0 ^ hz1
                e10 = hy1 ^ hz0
                e11 = hy1 ^ hz1
                x1 = xi + 1
                hs = (xi ^ e00, xi ^ e01, xi ^ e10, xi ^ e11,
                      x1 ^ e00, x1 ^ e01, x1 ^ e10, x1 ^ e11)
                base = g * 128
                for c in range(8):
                    t = hs[c] & MASK
                    d = base_l + (((t >> 7) << 8) | (t & 127))
                    ib[pl.ds(base + c * 16, 16)] = d
                    ib[pl.ds(NIDX + base + c * 16, 16)] = d + 128
                return carry

            lax.fori_loop(0, GRP, g_body, None)

        def fire(ib, fb, sem):
            return [pltpu.async_copy(
                tabs.at[ib.at[pl.ds(j * STR, STR)]],
                fb.at[pl.ds(j * STR, STR)], sem) for j in range(NSTR)]

        def interp(l, fb):
            res = float(RES[l])

            def g_body(g, carry):
                _, _, _, fx, fy, fz = _frac_parts(xyz, g * 16, res)
                wx0 = 1.0 - fx
                wy0 = 1.0 - fy
                wz0 = 1.0 - fz
                wxy = (wx0 * wy0, wx0 * fy, fx * wy0, fx * fy)
                acc0 = jnp.zeros((16,), jnp.float32)
                acc1 = jnp.zeros((16,), jnp.float32)
                base = g * 128
                for c in range(8):
                    w = wxy[c >> 1] * (fz if (c & 1) else wz0)
                    f0 = fb[pl.ds(base + c * 16, 16)]
                    f1 = fb[pl.ds(NIDX + base + c * 16, 16)]
                    acc0 = acc0 + w * f0
                    acc1 = acc1 + w * f1
                ob[2 * l, pl.ds(g * 16, 16)] = acc0
                ob[2 * l + 1, pl.ds(g * 16, 16)] = acc1
                return carry

            lax.fori_loop(0, GRP, g_body, None)

        def block(b, carry):
            col = wid * PPW + b * BLK
            pltpu.sync_copy(xt.at[:, pl.ds(col, BLK)], xyz)
            hash_level(0, ibufs[0])
            prev = fire(ibufs[0], fbufs[0], sems[0])
            for l in range(1, L):
                cb = l % 2
                pb = (l - 1) % 2
                hash_level(l, ibufs[cb])
                cur = fire(ibufs[cb], fbufs[cb], sems[cb])
                for cp in prev:
                    cp.wait()
                interp(l - 1, fbufs[pb])
                prev = cur
            for cp in prev:
                cp.wait()
            interp(L - 1, fbufs[(L - 1) % 2])
            pltpu.sync_copy(ob, out.at[:, pl.ds(col, BLK)])
            return carry

        lax.fori_loop(0, NBLK, block, None)

    return enc


_encode = _make_encoder()

TB = 2048


def _dot(a, b):
    return lax.dot_general(a, b, (((1,), (0,)), ((), ())),
                           preferred_element_type=jnp.float32)


def _mlp_body(f_ref, vt_ref, w1t_ref, b1_ref, w2t_ref, b2_ref, rwxt_ref,
              emat_ref, rwvt_ref, rwst_ref, rwct_ref, rb1_ref, rw2t_ref,
              rb2_ref, rw3t_ref, rb3_ref, sig_ref, rgb_ref):
    f = f_ref[...]
    h1 = jnp.maximum(_dot(w1t_ref[...], f) + b1_ref[...], 0.0)
    out = _dot(w2t_ref[...], h1) + b2_ref[...]
    sig_ref[...] = out[0:1, :]

    vt = vt_ref[...]
    v12 = _dot(emat_ref[...], vt)
    r = _dot(rwxt_ref[...], out) + rb1_ref[...]
    r = r + _dot(rwvt_ref[...], vt)
    r = r + _dot(rwst_ref[...], jnp.sin(v12))
    r = r + _dot(rwct_ref[...], jnp.cos(v12))
    h = jnp.maximum(r, 0.0)
    h = jnp.maximum(_dot(rw2t_ref[...], h) + rb2_ref[...], 0.0)
    o = _dot(rw3t_ref[...], h) + rb3_ref[...]
    rgb_ref[...] = 1.0 / (1.0 + jnp.exp(-o))


def _full(shape):
    return pl.BlockSpec(shape, lambda i: (0, 0))


_mlp = pl.pallas_call(
    _mlp_body,
    grid=(NPTS // TB,),
    in_specs=[
        pl.BlockSpec((2 * L, TB), lambda i: (0, i)),
        pl.BlockSpec((3, TB), lambda i: (0, i)),
        _full((64, 2 * L)),
        _full((64, 1)),
        _full((16, 64)),
        _full((16, 1)),
        _full((64, 16)),
        _full((12, 3)),
        _full((64, 3)),
        _full((64, 12)),
        _full((64, 12)),
        _full((64, 1)),
        _full((64, 64)),
        _full((64, 1)),
        _full((3, 64)),
        _full((3, 1)),
    ],
    out_specs=[
        pl.BlockSpec((1, TB), lambda i: (0, i)),
        pl.BlockSpec((3, TB), lambda i: (0, i)),
    ],
    out_shape=[
        jax.ShapeDtypeStruct((1, NPTS), jnp.float32),
        jax.ShapeDtypeStruct((3, NPTS), jnp.float32),
    ],
)

_SIN_ROWS = np.array([3 + 6 * k + d for k in range(4) for d in range(3)])
_COS_ROWS = _SIN_ROWS + 3
_EMAT = np.zeros((12, 3), np.float32)
for _k in range(4):
    for _d in range(3):
        _EMAT[3 * _k + _d, _d] = float(2.0 ** _k)


def kernel(x, v, tables, sw1, sb1, sw2, sb2, rw1, rb1, rw2, rb2, rw3, rb3):
    xt = x.T
    tabs = tables.reshape(L, T // 128, 128, F).transpose(0, 1, 3, 2).reshape(-1)
    feats = _encode(xt, tabs)
    rwx = jnp.concatenate(
        [jnp.zeros((1, 64), jnp.float32), rw1[27:42, :]], axis=0)
    sig, rgbt = _mlp(
        feats, v.T, sw1.T, sb1.reshape(-1, 1), sw2.T, sb2.reshape(-1, 1),
        rwx.T, jnp.asarray(_EMAT), rw1[0:3, :].T,
        jnp.take(rw1, _SIN_ROWS, axis=0).T,
        jnp.take(rw1, _COS_ROWS, axis=0).T,
        rb1.reshape(-1, 1), rw2.T, rb2.reshape(-1, 1), rw3.T,
        rb3.reshape(-1, 1))
    return sig.reshape(-1), rgbt.T


# speedup vs baseline: 254.5869x; 2.4884x over previous
import functools

import numpy as np
import jax
import jax.numpy as jnp
from jax import lax
from jax.experimental import pallas as pl
from jax.experimental.pallas import tpu as pltpu
from jax.experimental.pallas import tpu_sc as plsc

L = 16
F = 2
T = 1 << 19
N_MIN = 16
N_MAX = 2048
NPTS = 131072
_BG = float(np.exp((np.log(N_MAX) - np.log(N_MIN)) / (L - 1)))
RES = [int(np.floor(N_MIN * (_BG ** l))) for l in range(L)]
P2 = np.int32(2654435761 - (1 << 32))
P3 = np.int32(805459861)
MASK = np.int32(T - 1)

_info = plsc.get_sparse_core_info()
NC = _info.num_cores
NS = _info.num_subcores
NW = NC * NS
PPW = NPTS // NW
BLK = 128
NBLK = PPW // BLK
GRP = BLK // 16
NIDX = BLK * 8
STR = 512
NSTR = (NIDX * F) // STR


def _frac_parts(xyz, off, res):
    xv = xyz[0, pl.ds(off, 16)]
    yv = xyz[1, pl.ds(off, 16)]
    zv = xyz[2, pl.ds(off, 16)]
    xs = xv * res
    ys = yv * res
    zs = zv * res
    xi = xs.astype(jnp.int32)
    yi = ys.astype(jnp.int32)
    zi = zs.astype(jnp.int32)
    fx = xs - xi.astype(jnp.float32)
    fy = ys - yi.astype(jnp.float32)
    fz = zs - zi.astype(jnp.float32)
    return xi, yi, zi, fx, fy, fz


LVL_ELEMS = T * F
STAGE = LVL_ELEMS // 16


def _make_encoder():
    mesh = plsc.VectorSubcoreMesh(core_axis_name="c", subcore_axis_name="s")

    @functools.partial(
        pl.kernel,
        mesh=mesh,
        out_type=jax.ShapeDtypeStruct((2 * L, NPTS), jnp.float32),
        scratch_types=[
            pltpu.VMEM((3, PPW), jnp.float32),
            pltpu.VMEM((NIDX * F,), jnp.int32),
            pltpu.VMEM((NIDX * F,), jnp.int32),
            pltpu.VMEM((NIDX * F,), jnp.float32),
            pltpu.VMEM((NIDX * F,), jnp.float32),
            pltpu.VMEM((2, PPW), jnp.float32),
            pltpu.VMEM_SHARED((LVL_ELEMS,), jnp.float32),
            pltpu.SemaphoreType.DMA,
            pltpu.SemaphoreType.DMA,
            pltpu.SemaphoreType.DMA,
        ],
    )
    def enc(xt, tabs, out, xyz, ib0, ib1, fb0, fb1, obl, spm,
            sema, semb, semc):
        wid = lax.axis_index("s") * NC + lax.axis_index("c")
        sid = lax.axis_index("s")
        col0 = wid * PPW
        pltpu.sync_copy(xt.at[:, pl.ds(col0, PPW)], xyz)

        def bcast16(l):
            lv = jnp.zeros((16,), jnp.int32) + l
            r = jnp.zeros((16,), jnp.float32)
            for k in range(L):
                r = jnp.where(lv == k, jnp.float32(RES[k]), r)
            return r

        def parts(b, g, res):
            off = b * BLK + g * 16
            xv = xyz[0, pl.ds(off, 16)]
            yv = xyz[1, pl.ds(off, 16)]
            zv = xyz[2, pl.ds(off, 16)]
            xs = xv * res
            ys = yv * res
            zs = zv * res
            xi = xs.astype(jnp.int32)
            yi = ys.astype(jnp.int32)
            zi = zs.astype(jnp.int32)
            fx = xs - xi.astype(jnp.float32)
            fy = ys - yi.astype(jnp.float32)
            fz = zs - zi.astype(jnp.float32)
            return xi, yi, zi, fx, fy, fz

        def hash_blk(b, res, ib):
            def g_body(g, carry):
                xi, yi, zi, fx, fy, fz = parts(b, g, res)
                hy0 = yi * P2
                hy1 = hy0 + P2
                hz0 = zi * P3
                hz1 = hz0 + P3
                e00 = hy0 ^ hz0
                e01 = hy0 ^ hz1
                e10 = hy1 ^ hz0
                e11 = hy1 ^ hz1
                x1 = xi + 1
                hs = (xi ^ e00, xi ^ e01, xi ^ e10, xi ^ e11,
                      x1 ^ e00, x1 ^ e01, x1 ^ e10, x1 ^ e11)
                base = g * 128
                for c in range(8):
                    t = hs[c] & MASK
                    d = ((t >> 7) << 8) | (t & 127)
                    ib[pl.ds(base + c * 16, 16)] = d
                    ib[pl.ds(NIDX + base + c * 16, 16)] = d + 128
                return carry

            lax.fori_loop(0, GRP, g_body, None)

        def fire(ib, fb, sem):
            for j in range(NSTR):
                pltpu.async_copy(spm.at[ib.at[pl.ds(j * STR, STR)]],
                                 fb.at[pl.ds(j * STR, STR)], sem)

        def drain_fb(fb, sem):
            pltpu.make_async_copy(tabs.at[pl.ds(0, NIDX * F)], fb, sem).wait()

        def interp(b, res, fb):
            def g_body(g, carry):
                _, _, _, fx, fy, fz = parts(b, g, res)
                wx0 = 1.0 - fx
                wy0 = 1.0 - fy
                wz0 = 1.0 - fz
                wxy = (wx0 * wy0, wx0 * fy, fx * wy0, fx * fy)
                acc0 = jnp.zeros((16,), jnp.float32)
                acc1 = jnp.zeros((16,), jnp.float32)
                base = g * 128
                for c in range(8):
                    w = wxy[c >> 1] * (fz if (c & 1) else wz0)
                    f0 = fb[pl.ds(base + c * 16, 16)]
                    f1 = fb[pl.ds(NIDX + base + c * 16, 16)]
                    acc0 = acc0 + w * f0
                    acc1 = acc1 + w * f1
                off = b * BLK + g * 16
                obl[0, pl.ds(off, 16)] = acc0
                obl[1, pl.ds(off, 16)] = acc1
                return carry

            lax.fori_loop(0, GRP, g_body, None)

        def level(l, carry):
            plsc.subcore_barrier()
            pltpu.sync_copy(
                tabs.at[pl.ds(l * LVL_ELEMS + sid * STAGE, STAGE)],
                spm.at[pl.ds(sid * STAGE, STAGE)])
            plsc.subcore_barrier()
            res = bcast16(l)

            @pl.when(l > 0)
            def _():
                pltpu.make_async_copy(
                    out.at[pl.ds(0, 2), pl.ds(0, PPW)], obl, semc).wait()

            hash_blk(0, res, ib0)
            fire(ib0, fb0, sema)

            def super_body(k, carry2):
                hash_blk(2 * k + 1, res, ib1)
                fire(ib1, fb1, semb)
                drain_fb(fb0, sema)
                interp(2 * k, res, fb0)
                hash_blk(2 * k + 2, res, ib0)
                fire(ib0, fb0, sema)
                drain_fb(fb1, semb)
                interp(2 * k + 1, res, fb1)
                return carry2

            lax.fori_loop(0, NBLK // 2 - 1, super_body, None)
            hash_blk(NBLK - 1, res, ib1)
            fire(ib1, fb1, semb)
            drain_fb(fb0, sema)
            interp(NBLK - 2, res, fb0)
            drain_fb(fb1, semb)
            interp(NBLK - 1, res, fb1)
            pltpu.async_copy(
                obl, out.at[pl.ds(2 * l, 2), pl.ds(col0, PPW)], semc)
            return carry

        lax.fori_loop(0, L, level, None)
        pltpu.make_async_copy(
            out.at[pl.ds(0, 2), pl.ds(0, PPW)], obl, semc).wait()

    return enc


_encode = _make_encoder()

TB = 2048


def _dot(a, b):
    return lax.dot_general(a, b, (((1,), (0,)), ((), ())),
                           preferred_element_type=jnp.float32)


def _mlp_body(f_ref, vt_ref, w1t_ref, b1_ref, w2t_ref, b2_ref, rwxt_ref,
              emat_ref, rwvt_ref, rwst_ref, rwct_ref, rb1_ref, rw2t_ref,
              rb2_ref, rw3t_ref, rb3_ref, sig_ref, rgb_ref):
    f = f_ref[...]
    h1 = jnp.maximum(_dot(w1t_ref[...], f) + b1_ref[...], 0.0)
    out = _dot(w2t_ref[...], h1) + b2_ref[...]
    sig_ref[...] = out[0:1, :]

    vt = vt_ref[...]
    v12 = _dot(emat_ref[...], vt)
    r = _dot(rwxt_ref[...], out) + rb1_ref[...]
    r = r + _dot(rwvt_ref[...], vt)
    r = r + _dot(rwst_ref[...], jnp.sin(v12))
    r = r + _dot(rwct_ref[...], jnp.cos(v12))
    h = jnp.maximum(r, 0.0)
    h = jnp.maximum(_dot(rw2t_ref[...], h) + rb2_ref[...], 0.0)
    o = _dot(rw3t_ref[...], h) + rb3_ref[...]
    rgb_ref[...] = 1.0 / (1.0 + jnp.exp(-o))


def _full(shape):
    return pl.BlockSpec(shape, lambda i: (0, 0))


_mlp = pl.pallas_call(
    _mlp_body,
    grid=(NPTS // TB,),
    in_specs=[
        pl.BlockSpec((2 * L, TB), lambda i: (0, i)),
        pl.BlockSpec((3, TB), lambda i: (0, i)),
        _full((64, 2 * L)),
        _full((64, 1)),
        _full((16, 64)),
        _full((16, 1)),
        _full((64, 16)),
        _full((12, 3)),
        _full((64, 3)),
        _full((64, 12)),
        _full((64, 12)),
        _full((64, 1)),
        _full((64, 64)),
        _full((64, 1)),
        _full((3, 64)),
        _full((3, 1)),
    ],
    out_specs=[
        pl.BlockSpec((1, TB), lambda i: (0, i)),
        pl.BlockSpec((3, TB), lambda i: (0, i)),
    ],
    out_shape=[
        jax.ShapeDtypeStruct((1, NPTS), jnp.float32),
        jax.ShapeDtypeStruct((3, NPTS), jnp.float32),
    ],
)

_SIN_ROWS = np.array([3 + 6 * k + d for k in range(4) for d in range(3)])
_COS_ROWS = _SIN_ROWS + 3
_EMAT = np.zeros((12, 3), np.float32)
for _k in range(4):
    for _d in range(3):
        _EMAT[3 * _k + _d, _d] = float(2.0 ** _k)


def kernel(x, v, tables, sw1, sb1, sw2, sb2, rw1, rb1, rw2, rb2, rw3, rb3):
    xt = x.T
    tabs = tables.reshape(L, T // 128, 128, F).transpose(0, 1, 3, 2).reshape(-1)
    feats = _encode(xt, tabs)
    rwx = jnp.concatenate(
        [jnp.zeros((1, 64), jnp.float32), rw1[27:42, :]], axis=0)
    sig, rgbt = _mlp(
        feats, v.T, sw1.T, sb1.reshape(-1, 1), sw2.T, sb2.reshape(-1, 1),
        rwx.T, jnp.asarray(_EMAT), rw1[0:3, :].T,
        jnp.take(rw1, _SIN_ROWS, axis=0).T,
        jnp.take(rw1, _COS_ROWS, axis=0).T,
        rb1.reshape(-1, 1), rw2.T, rb2.reshape(-1, 1), rw3.T,
        rb3.reshape(-1, 1))
    return sig.reshape(-1), rgbt.T


# speedup vs baseline: 281.5052x; 1.1057x over previous
import functools

import numpy as np
import jax
import jax.numpy as jnp
from jax import lax
from jax.experimental import pallas as pl
from jax.experimental.pallas import tpu as pltpu
from jax.experimental.pallas import tpu_sc as plsc

L = 16
F = 2
T = 1 << 19
N_MIN = 16
N_MAX = 2048
NPTS = 131072
_BG = float(np.exp((np.log(N_MAX) - np.log(N_MIN)) / (L - 1)))
RES = [int(np.floor(N_MIN * (_BG ** l))) for l in range(L)]
P2 = np.int32(2654435761 - (1 << 32))
P3 = np.int32(805459861)
MASK = np.int32(T - 1)

_info = plsc.get_sparse_core_info()
NC = _info.num_cores
NS = _info.num_subcores
NW = NC * NS
PPW = NPTS // NS
BLK = 128
NBLK = PPW // BLK
GRP = BLK // 16
NIDX = BLK * 8
STR = 512
NSTR = (NIDX * F) // STR


def _frac_parts(xyz, off, res):
    xv = xyz[0, pl.ds(off, 16)]
    yv = xyz[1, pl.ds(off, 16)]
    zv = xyz[2, pl.ds(off, 16)]
    xs = xv * res
    ys = yv * res
    zs = zv * res
    xi = xs.astype(jnp.int32)
    yi = ys.astype(jnp.int32)
    zi = zs.astype(jnp.int32)
    fx = xs - xi.astype(jnp.float32)
    fy = ys - yi.astype(jnp.float32)
    fz = zs - zi.astype(jnp.float32)
    return xi, yi, zi, fx, fy, fz


LVL_ELEMS = T * F
STAGE = LVL_ELEMS // 16


def _make_encoder():
    mesh = plsc.VectorSubcoreMesh(core_axis_name="c", subcore_axis_name="s")

    @functools.partial(
        pl.kernel,
        mesh=mesh,
        out_type=jax.ShapeDtypeStruct((2 * L, NPTS), jnp.float32),
        scratch_types=[
            pltpu.VMEM((3, PPW), jnp.float32),
            pltpu.VMEM((NIDX * F,), jnp.int32),
            pltpu.VMEM((NIDX * F,), jnp.int32),
            pltpu.VMEM((NIDX * F,), jnp.float32),
            pltpu.VMEM((NIDX * F,), jnp.float32),
            pltpu.VMEM((2, PPW), jnp.float32),
            pltpu.VMEM_SHARED((LVL_ELEMS,), jnp.float32),
            pltpu.SemaphoreType.DMA,
            pltpu.SemaphoreType.DMA,
            pltpu.SemaphoreType.DMA,
        ],
    )
    def enc(xt, tabs, out, xyz, ib0, ib1, fb0, fb1, obl, spm,
            sema, semb, semc):
        cid = lax.axis_index("c")
        sid = lax.axis_index("s")
        col0 = sid * PPW
        pltpu.sync_copy(xt.at[:, pl.ds(col0, PPW)], xyz)

        def bcast16(l):
            lv = jnp.zeros((16,), jnp.int32) + l
            r = jnp.zeros((16,), jnp.float32)
            for k in range(L):
                r = jnp.where(lv == k, jnp.float32(RES[k]), r)
            return r

        def parts(b, g, res):
            off = b * BLK + g * 16
            xv = xyz[0, pl.ds(off, 16)]
            yv = xyz[1, pl.ds(off, 16)]
            zv = xyz[2, pl.ds(off, 16)]
            xs = xv * res
            ys = yv * res
            zs = zv * res
            xi = xs.astype(jnp.int32)
            yi = ys.astype(jnp.int32)
            zi = zs.astype(jnp.int32)
            fx = xs - xi.astype(jnp.float32)
            fy = ys - yi.astype(jnp.float32)
            fz = zs - zi.astype(jnp.float32)
            return xi, yi, zi, fx, fy, fz

        def hash_blk(b, res, ib):
            def g_body(g, carry):
                xi, yi, zi, fx, fy, fz = parts(b, g, res)
                hy0 = yi * P2
                hy1 = hy0 + P2
                hz0 = zi * P3
                hz1 = hz0 + P3
                e00 = hy0 ^ hz0
                e01 = hy0 ^ hz1
                e10 = hy1 ^ hz0
                e11 = hy1 ^ hz1
                x1 = xi + 1
                hs = (xi ^ e00, xi ^ e01, xi ^ e10, xi ^ e11,
                      x1 ^ e00, x1 ^ e01, x1 ^ e10, x1 ^ e11)
                base = g * 128
                for c in range(8):
                    t = hs[c] & MASK
                    d = ((t >> 7) << 8) | (t & 127)
                    ib[pl.ds(base + c * 16, 16)] = d
                    ib[pl.ds(NIDX + base + c * 16, 16)] = d + 128
                return carry

            lax.fori_loop(0, GRP, g_body, None)

        def fire(ib, fb, sem):
            for j in range(NSTR):
                pltpu.async_copy(spm.at[ib.at[pl.ds(j * STR, STR)]],
                                 fb.at[pl.ds(j * STR, STR)], sem)

        def drain_fb(fb, sem):
            pltpu.make_async_copy(tabs.at[pl.ds(0, NIDX * F)], fb, sem).wait()

        def interp(b, res, fb):
            def g_body(g, carry):
                _, _, _, fx, fy, fz = parts(b, g, res)
                wx0 = 1.0 - fx
                wy0 = 1.0 - fy
                wz0 = 1.0 - fz
                wxy = (wx0 * wy0, wx0 * fy, fx * wy0, fx * fy)
                acc0 = jnp.zeros((16,), jnp.float32)
                acc1 = jnp.zeros((16,), jnp.float32)
                base = g * 128
                for c in range(8):
                    w = wxy[c >> 1] * (fz if (c & 1) else wz0)
                    f0 = fb[pl.ds(base + c * 16, 16)]
                    f1 = fb[pl.ds(NIDX + base + c * 16, 16)]
                    acc0 = acc0 + w * f0
                    acc1 = acc1 + w * f1
                off = b * BLK + g * 16
                obl[0, pl.ds(off, 16)] = acc0
                obl[1, pl.ds(off, 16)] = acc1
                return carry

            lax.fori_loop(0, GRP, g_body, None)

        def level(l, carry):
            la = cid * (L // 2) + l
            plsc.subcore_barrier()
            pltpu.sync_copy(
                tabs.at[pl.ds(la * LVL_ELEMS + sid * STAGE, STAGE)],
                spm.at[pl.ds(sid * STAGE, STAGE)])
            plsc.subcore_barrier()
            res = bcast16(la)

            @pl.when(l > 0)
            def _():
                pltpu.make_async_copy(
                    out.at[pl.ds(0, 2), pl.ds(0, PPW)], obl, semc).wait()

            hash_blk(0, res, ib0)
            fire(ib0, fb0, sema)

            def super_body(k, carry2):
                hash_blk(2 * k + 1, res, ib1)
                fire(ib1, fb1, semb)
                drain_fb(fb0, sema)
                interp(2 * k, res, fb0)
                hash_blk(2 * k + 2, res, ib0)
                fire(ib0, fb0, sema)
                drain_fb(fb1, semb)
                interp(2 * k + 1, res, fb1)
                return carry2

            lax.fori_loop(0, NBLK // 2 - 1, super_body, None)
            hash_blk(NBLK - 1, res, ib1)
            fire(ib1, fb1, semb)
            drain_fb(fb0, sema)
            interp(NBLK - 2, res, fb0)
            drain_fb(fb1, semb)
            interp(NBLK - 1, res, fb1)
            pltpu.async_copy(
                obl, out.at[pl.ds(2 * la, 2), pl.ds(col0, PPW)], semc)
            return carry

        lax.fori_loop(0, L // 2, level, None)
        pltpu.make_async_copy(
            out.at[pl.ds(0, 2), pl.ds(0, PPW)], obl, semc).wait()

    return enc


_encode = _make_encoder()

TB = 2048


def _dot(a, b):
    return lax.dot_general(a, b, (((1,), (0,)), ((), ())),
                           preferred_element_type=jnp.float32)


def _mlp_body(f_ref, vt_ref, w1t_ref, b1_ref, w2t_ref, b2_ref, rwxt_ref,
              emat_ref, rwvt_ref, rwst_ref, rwct_ref, rb1_ref, rw2t_ref,
              rb2_ref, rw3t_ref, rb3_ref, sig_ref, rgb_ref):
    f = f_ref[...]
    h1 = jnp.maximum(_dot(w1t_ref[...], f) + b1_ref[...], 0.0)
    out = _dot(w2t_ref[...], h1) + b2_ref[...]
    sig_ref[...] = out[0:1, :]

    vt = vt_ref[...]
    v12 = _dot(emat_ref[...], vt)
    r = _dot(rwxt_ref[...], out) + rb1_ref[...]
    r = r + _dot(rwvt_ref[...], vt)
    r = r + _dot(rwst_ref[...], jnp.sin(v12))
    r = r + _dot(rwct_ref[...], jnp.cos(v12))
    h = jnp.maximum(r, 0.0)
    h = jnp.maximum(_dot(rw2t_ref[...], h) + rb2_ref[...], 0.0)
    o = _dot(rw3t_ref[...], h) + rb3_ref[...]
    rgb_ref[...] = 1.0 / (1.0 + jnp.exp(-o))


def _full(shape):
    return pl.BlockSpec(shape, lambda i: (0, 0))


_mlp = pl.pallas_call(
    _mlp_body,
    grid=(NPTS // TB,),
    in_specs=[
        pl.BlockSpec((2 * L, TB), lambda i: (0, i)),
        pl.BlockSpec((3, TB), lambda i: (0, i)),
        _full((64, 2 * L)),
        _full((64, 1)),
        _full((16, 64)),
        _full((16, 1)),
        _full((64, 16)),
        _full((12, 3)),
        _full((64, 3)),
        _full((64, 12)),
        _full((64, 12)),
        _full((64, 1)),
        _full((64, 64)),
        _full((64, 1)),
        _full((3, 64)),
        _full((3, 1)),
    ],
    out_specs=[
        pl.BlockSpec((1, TB), lambda i: (0, i)),
        pl.BlockSpec((3, TB), lambda i: (0, i)),
    ],
    out_shape=[
        jax.ShapeDtypeStruct((1, NPTS), jnp.float32),
        jax.ShapeDtypeStruct((3, NPTS), jnp.float32),
    ],
)

_SIN_ROWS = np.array([3 + 6 * k + d for k in range(4) for d in range(3)])
_COS_ROWS = _SIN_ROWS + 3
_EMAT = np.zeros((12, 3), np.float32)
for _k in range(4):
    for _d in range(3):
        _EMAT[3 * _k + _d, _d] = float(2.0 ** _k)


def kernel(x, v, tables, sw1, sb1, sw2, sb2, rw1, rb1, rw2, rb2, rw3, rb3):
    xt = x.T
    tabs = tables.reshape(L, T // 128, 128, F).transpose(0, 1, 3, 2).reshape(-1)
    feats = _encode(xt, tabs)
    rwx = jnp.concatenate(
        [jnp.zeros((1, 64), jnp.float32), rw1[27:42, :]], axis=0)
    sig, rgbt = _mlp(
        feats, v.T, sw1.T, sb1.reshape(-1, 1), sw2.T, sb2.reshape(-1, 1),
        rwx.T, jnp.asarray(_EMAT), rw1[0:3, :].T,
        jnp.take(rw1, _SIN_ROWS, axis=0).T,
        jnp.take(rw1, _COS_ROWS, axis=0).T,
        rb1.reshape(-1, 1), rw2.T, rb2.reshape(-1, 1), rw3.T,
        rb3.reshape(-1, 1))
    return sig.reshape(-1), rgbt.T


# speedup vs baseline: 297.2508x; 1.0559x over previous
import functools

import numpy as np
import jax
import jax.numpy as jnp
from jax import lax
from jax.experimental import pallas as pl
from jax.experimental.pallas import tpu as pltpu
from jax.experimental.pallas import tpu_sc as plsc

L = 16
F = 2
T = 1 << 19
N_MIN = 16
N_MAX = 2048
NPTS = 131072
_BG = float(np.exp((np.log(N_MAX) - np.log(N_MIN)) / (L - 1)))
RES = [int(np.floor(N_MIN * (_BG ** l))) for l in range(L)]
P2 = np.int32(2654435761 - (1 << 32))
P3 = np.int32(805459861)
MASK = np.int32(T - 1)

_info = plsc.get_sparse_core_info()
NC = _info.num_cores
NS = _info.num_subcores
NW = NC * NS
PPW = NPTS // NS
BLK = 128
NBLK = PPW // BLK
GRP = BLK // 16
NIDX = BLK * 8
STR = 512
NSTR = (NIDX * F) // STR


def _frac_parts(xyz, off, res):
    xv = xyz[0, pl.ds(off, 16)]
    yv = xyz[1, pl.ds(off, 16)]
    zv = xyz[2, pl.ds(off, 16)]
    xs = xv * res
    ys = yv * res
    zs = zv * res
    xi = xs.astype(jnp.int32)
    yi = ys.astype(jnp.int32)
    zi = zs.astype(jnp.int32)
    fx = xs - xi.astype(jnp.float32)
    fy = ys - yi.astype(jnp.float32)
    fz = zs - zi.astype(jnp.float32)
    return xi, yi, zi, fx, fy, fz


LVL_ELEMS = T * F
STAGE = LVL_ELEMS // 16


def _make_encoder():
    mesh = plsc.VectorSubcoreMesh(core_axis_name="c", subcore_axis_name="s")

    @functools.partial(
        pl.kernel,
        mesh=mesh,
        out_type=jax.ShapeDtypeStruct((2 * L, NPTS), jnp.float32),
        scratch_types=[
            pltpu.VMEM((3, PPW), jnp.float32),
            pltpu.VMEM((NIDX * F,), jnp.int32),
            pltpu.VMEM((NIDX * F,), jnp.int32),
            pltpu.VMEM((NIDX * F,), jnp.float32),
            pltpu.VMEM((NIDX * F,), jnp.float32),
            pltpu.VMEM((2, PPW), jnp.float32),
            pltpu.VMEM_SHARED((LVL_ELEMS,), jnp.float32),
            pltpu.SemaphoreType.DMA,
            pltpu.SemaphoreType.DMA,
            pltpu.SemaphoreType.DMA,
        ],
    )
    def enc(xt, tabs, out, xyz, ib0, ib1, fb0, fb1, obl, spm,
            sema, semb, semc):
        cid = lax.axis_index("c")
        sid = lax.axis_index("s")
        col0 = sid * PPW
        pltpu.sync_copy(xt.at[:, pl.ds(col0, PPW)], xyz)

        def bcast16(l):
            lv = jnp.zeros((16,), jnp.int32) + l
            r = jnp.zeros((16,), jnp.float32)
            for k in range(L):
                r = jnp.where(lv == k, jnp.float32(RES[k]), r)
            return r

        def parts(b, g, res):
            off = b * BLK + g * 16
            xv = xyz[0, pl.ds(off, 16)]
            yv = xyz[1, pl.ds(off, 16)]
            zv = xyz[2, pl.ds(off, 16)]
            xs = xv * res
            ys = yv * res
            zs = zv * res
            xi = xs.astype(jnp.int32)
            yi = ys.astype(jnp.int32)
            zi = zs.astype(jnp.int32)
            fx = xs - xi.astype(jnp.float32)
            fy = ys - yi.astype(jnp.float32)
            fz = zs - zi.astype(jnp.float32)
            return xi, yi, zi, fx, fy, fz

        def hash_blk(b, res, ib):
            def g_body(g, carry):
                xi, yi, zi, fx, fy, fz = parts(b, g, res)
                hy0 = yi * P2
                hy1 = hy0 + P2
                hz0 = zi * P3
                hz1 = hz0 + P3
                e00 = hy0 ^ hz0
                e01 = hy0 ^ hz1
                e10 = hy1 ^ hz0
                e11 = hy1 ^ hz1
                x1 = xi + 1
                hs = (xi ^ e00, xi ^ e01, xi ^ e10, xi ^ e11,
                      x1 ^ e00, x1 ^ e01, x1 ^ e10, x1 ^ e11)
                base = g * 128
                for c in range(8):
                    t = hs[c] & MASK
                    d = ((t >> 7) << 8) | (t & 127)
                    ib[pl.ds(base + c * 16, 16)] = d
                    ib[pl.ds(NIDX + base + c * 16, 16)] = d + 128
                return carry

            lax.fori_loop(0, GRP, g_body, None)

        def fire(ib, fb, sem):
            for j in range(NSTR):
                pltpu.async_copy(spm.at[ib.at[pl.ds(j * STR, STR)]],
                                 fb.at[pl.ds(j * STR, STR)], sem)

        def drain_fb(fb, sem):
            pltpu.make_async_copy(tabs.at[pl.ds(0, NIDX * F)], fb, sem).wait()

        def interp(b, res, fb):
            def g_body(g, carry):
                _, _, _, fx, fy, fz = parts(b, g, res)
                wx0 = 1.0 - fx
                wy0 = 1.0 - fy
                wz0 = 1.0 - fz
                wxy = (wx0 * wy0, wx0 * fy, fx * wy0, fx * fy)
                acc0 = jnp.zeros((16,), jnp.float32)
                acc1 = jnp.zeros((16,), jnp.float32)
                base = g * 128
                for c in range(8):
                    w = wxy[c >> 1] * (fz if (c & 1) else wz0)
                    f0 = fb[pl.ds(base + c * 16, 16)]
                    f1 = fb[pl.ds(NIDX + base + c * 16, 16)]
                    acc0 = acc0 + w * f0
                    acc1 = acc1 + w * f1
                off = b * BLK + g * 16
                obl[0, pl.ds(off, 16)] = acc0
                obl[1, pl.ds(off, 16)] = acc1
                return carry

            lax.fori_loop(0, GRP, g_body, None)

        def level(l, carry):
            la = cid * (L // 2) + l
            plsc.subcore_barrier()
            pltpu.sync_copy(
                tabs.at[pl.ds(la * LVL_ELEMS + sid * STAGE, STAGE)],
                spm.at[pl.ds(sid * STAGE, STAGE)])
            plsc.subcore_barrier()
            res = bcast16(la)

            @pl.when(l > 0)
            def _():
                pltpu.make_async_copy(
                    out.at[pl.ds(0, 2), pl.ds(0, PPW)], obl, semc).wait()

            hash_blk(0, res, ib0)
            fire(ib0, fb0, sema)

            def super_body(k, carry2):
                hash_blk(2 * k + 1, res, ib1)
                fire(ib1, fb1, semb)
                drain_fb(fb0, sema)
                interp(2 * k, res, fb0)
                hash_blk(2 * k + 2, res, ib0)
                fire(ib0, fb0, sema)
                drain_fb(fb1, semb)
                interp(2 * k + 1, res, fb1)
                return carry2

            lax.fori_loop(0, NBLK // 2 - 1, super_body, None)
            hash_blk(NBLK - 1, res, ib1)
            fire(ib1, fb1, semb)
            drain_fb(fb0, sema)
            interp(NBLK - 2, res, fb0)
            drain_fb(fb1, semb)
            interp(NBLK - 1, res, fb1)
            pltpu.async_copy(
                obl, out.at[pl.ds(2 * la, 2), pl.ds(col0, PPW)], semc)
            return carry

        lax.fori_loop(0, L // 2, level, None)
        pltpu.make_async_copy(
            out.at[pl.ds(0, 2), pl.ds(0, PPW)], obl, semc).wait()

    return enc


_encode = _make_encoder()

TB = 4096


def _dot(a, b):
    return lax.dot_general(a, b, (((1,), (0,)), ((), ())),
                           preferred_element_type=jnp.float32)


def _mlp_body(f_ref, vt_ref, w1t_ref, b1_ref, w2t_ref, b2_ref, rwxt_ref,
              emat_ref, rwvt_ref, rwst_ref, rwct_ref, rb1_ref, rw2t_ref,
              rb2_ref, rw3t_ref, rb3_ref, sig_ref, rgb_ref):
    f = f_ref[...]
    h1 = jnp.maximum(_dot(w1t_ref[...], f) + b1_ref[...], 0.0)
    out = _dot(w2t_ref[...], h1) + b2_ref[...]
    sig_ref[...] = out[0:1, :]

    vt = vt_ref[...]
    v12 = _dot(emat_ref[...], vt)
    r = _dot(rwxt_ref[...], out) + rb1_ref[...]
    r = r + _dot(rwvt_ref[...], vt)
    r = r + _dot(rwst_ref[...], jnp.sin(v12))
    r = r + _dot(rwct_ref[...], jnp.cos(v12))
    h = jnp.maximum(r, 0.0)
    h = jnp.maximum(_dot(rw2t_ref[...], h) + rb2_ref[...], 0.0)
    o = _dot(rw3t_ref[...], h) + rb3_ref[...]
    rgb_ref[...] = 1.0 / (1.0 + jnp.exp(-o))


def _full(shape):
    return pl.BlockSpec(shape, lambda i: (0, 0))


_mlp = pl.pallas_call(
    _mlp_body,
    grid=(NPTS // TB,),
    in_specs=[
        pl.BlockSpec((2 * L, TB), lambda i: (0, i)),
        pl.BlockSpec((3, TB), lambda i: (0, i)),
        _full((64, 2 * L)),
        _full((64, 1)),
        _full((16, 64)),
        _full((16, 1)),
        _full((64, 16)),
        _full((12, 3)),
        _full((64, 3)),
        _full((64, 12)),
        _full((64, 12)),
        _full((64, 1)),
        _full((64, 64)),
        _full((64, 1)),
        _full((3, 64)),
        _full((3, 1)),
    ],
    out_specs=[
        pl.BlockSpec((1, TB), lambda i: (0, i)),
        pl.BlockSpec((3, TB), lambda i: (0, i)),
    ],
    out_shape=[
        jax.ShapeDtypeStruct((1, NPTS), jnp.float32),
        jax.ShapeDtypeStruct((3, NPTS), jnp.float32),
    ],
)

_SIN_ROWS = np.array([3 + 6 * k + d for k in range(4) for d in range(3)])
_COS_ROWS = _SIN_ROWS + 3
_EMAT = np.zeros((12, 3), np.float32)
for _k in range(4):
    for _d in range(3):
        _EMAT[3 * _k + _d, _d] = float(2.0 ** _k)


def kernel(x, v, tables, sw1, sb1, sw2, sb2, rw1, rb1, rw2, rb2, rw3, rb3):
    xt = x.T
    tabs = tables.reshape(L, T // 128, 128, F).transpose(0, 1, 3, 2).reshape(-1)
    feats = _encode(xt, tabs)
    rwx = jnp.concatenate(
        [jnp.zeros((1, 64), jnp.float32), rw1[27:42, :]], axis=0)
    sig, rgbt = _mlp(
        feats, v.T, sw1.T, sb1.reshape(-1, 1), sw2.T, sb2.reshape(-1, 1),
        rwx.T, jnp.asarray(_EMAT), rw1[0:3, :].T,
        jnp.take(rw1, _SIN_ROWS, axis=0).T,
        jnp.take(rw1, _COS_ROWS, axis=0).T,
        rb1.reshape(-1, 1), rw2.T, rb2.reshape(-1, 1), rw3.T,
        rb3.reshape(-1, 1))
    return sig.reshape(-1), rgbt.T
